# baseline (device time: 10208 ns/iter reference)
import jax
import jax.numpy as jnp
from jax import lax
from jax.experimental import pallas as pl
from jax.experimental.pallas import tpu as pltpu

N_DEV = 4


def kernel(x):
    m_per, n = x.shape

    def body(x_ref, out_ref, comm_ref, send_sems, recv_sems):
        my_pos = lax.axis_index("i")
        peers = [(my_pos + d) % N_DEV for d in range(1, N_DEV)]

        comm_ref[0, :, :] = jnp.sum(x_ref[:, :], axis=0, keepdims=True)
        rdmas = []
        for d in range(1, N_DEV):
            rdma = pltpu.make_async_remote_copy(
                src_ref=comm_ref.at[0],
                dst_ref=comm_ref.at[d],
                send_sem=send_sems.at[d - 1],
                recv_sem=recv_sems.at[d - 1],
                device_id=((my_pos + d) % N_DEV,),
                device_id_type=pl.DeviceIdType.MESH,
            )
            rdma.start()
            rdmas.append(rdma)

        row = lax.broadcasted_iota(jnp.int32, (m_per, m_per), 0)
        col = lax.broadcasted_iota(jnp.int32, (m_per, m_per), 1)
        tri = (col <= row).astype(jnp.float32)
        local = lax.dot_general(
            tri, x_ref[:, :],
            (((1,), (0,)), ((), ())),
            preferred_element_type=jnp.float32,
        )

        offset = jnp.zeros((1, n), jnp.float32)
        for d in range(1, N_DEV):
            rdmas[d - 1].wait_recv()
            origin = (my_pos - d) % N_DEV
            offset = offset + jnp.where(
                origin < my_pos, comm_ref[d, :, :], 0.0
            )
        out_ref[:, :] = local + offset

        for r in rdmas:
            r.wait_send()

    return pl.pallas_call(
        body,
        out_shape=jax.ShapeDtypeStruct((m_per, n), jnp.float32),
        in_specs=[pl.BlockSpec(memory_space=pltpu.VMEM)],
        out_specs=pl.BlockSpec(memory_space=pltpu.VMEM),
        scratch_shapes=[
            pltpu.VMEM((N_DEV, 1, n), jnp.float32),
            pltpu.SemaphoreType.DMA((N_DEV - 1,)),
            pltpu.SemaphoreType.DMA((N_DEV - 1,)),
        ],
        compiler_params=pltpu.CompilerParams(),
    )(x)


# device time: 6788 ns/iter; 1.5038x vs baseline; 1.5038x over previous
import jax
import jax.numpy as jnp
from jax import lax
from jax.experimental import pallas as pl
from jax.experimental.pallas import tpu as pltpu

N_DEV = 4


def kernel(x):
    m_per, n = x.shape

    def body(x_ref, out_ref, comm_ref, send_sems, recv_sems):
        my_pos = lax.axis_index("i")
        peers = [(my_pos + d) % N_DEV for d in range(1, N_DEV)]

        barrier_sem = pltpu.get_barrier_semaphore()
        for p in peers:
            pl.semaphore_signal(
                barrier_sem, inc=1,
                device_id=(p,), device_id_type=pl.DeviceIdType.MESH,
            )

        comm_ref[0, :, :] = jnp.sum(x_ref[:, :], axis=0, keepdims=True)

        row = lax.broadcasted_iota(jnp.int32, (m_per, m_per), 0)
        col = lax.broadcasted_iota(jnp.int32, (m_per, m_per), 1)
        tri = (col <= row).astype(jnp.float32)

        pl.semaphore_wait(barrier_sem, N_DEV - 1)
        rdmas = []
        for d in range(1, N_DEV):
            rdma = pltpu.make_async_remote_copy(
                src_ref=comm_ref.at[0],
                dst_ref=comm_ref.at[d],
                send_sem=send_sems.at[d - 1],
                recv_sem=recv_sems.at[d - 1],
                device_id=((my_pos + d) % N_DEV,),
                device_id_type=pl.DeviceIdType.MESH,
            )
            rdma.start()
            rdmas.append(rdma)

        local = lax.dot_general(
            tri, x_ref[:, :],
            (((1,), (0,)), ((), ())),
            preferred_element_type=jnp.float32,
        )

        offset = jnp.zeros((1, n), jnp.float32)
        for d in (1, 3, 2):
            rdmas[d - 1].wait_recv()
            origin = (my_pos - d) % N_DEV
            offset = offset + jnp.where(
                origin < my_pos, comm_ref[d, :, :], 0.0
            )
        out_ref[:, :] = local + offset

        for r in rdmas:
            r.wait_send()

    return pl.pallas_call(
        body,
        out_shape=jax.ShapeDtypeStruct((m_per, n), jnp.float32),
        in_specs=[pl.BlockSpec(memory_space=pltpu.VMEM)],
        out_specs=pl.BlockSpec(memory_space=pltpu.VMEM),
        scratch_shapes=[
            pltpu.VMEM((N_DEV, 1, n), jnp.float32),
            pltpu.SemaphoreType.DMA((N_DEV - 1,)),
            pltpu.SemaphoreType.DMA((N_DEV - 1,)),
        ],
        compiler_params=pltpu.CompilerParams(collective_id=0),
    )(x)


# device time: 6029 ns/iter; 1.6931x vs baseline; 1.1259x over previous
import jax
import jax.numpy as jnp
from jax import lax
from jax.experimental import pallas as pl
from jax.experimental.pallas import tpu as pltpu

N_DEV = 4


def kernel(x):
    m_per, n = x.shape

    def body(x_ref, out_ref, comm_ref, send_sems, recv_sems):
        my_pos = lax.axis_index("i")

        barrier_sem = pltpu.get_barrier_semaphore()
        for d in range(1, N_DEV):

            @pl.when(my_pos >= d)
            def _():
                pl.semaphore_signal(
                    barrier_sem, inc=1,
                    device_id=(my_pos - d,),
                    device_id_type=pl.DeviceIdType.MESH,
                )

        comm_ref[0, :, :] = jnp.sum(x_ref[:, :], axis=0, keepdims=True)
        row = lax.broadcasted_iota(jnp.int32, (m_per, m_per), 0)
        col = lax.broadcasted_iota(jnp.int32, (m_per, m_per), 1)
        tri = (col <= row).astype(jnp.float32)

        for d in range(1, N_DEV):

            @pl.when(my_pos + d <= N_DEV - 1)
            def _():
                pl.semaphore_wait(barrier_sem, 1)

        for d in range(1, N_DEV):

            @pl.when(my_pos + d <= N_DEV - 1)
            def _():
                pltpu.make_async_remote_copy(
                    src_ref=comm_ref.at[0],
                    dst_ref=comm_ref.at[d],
                    send_sem=send_sems.at[d - 1],
                    recv_sem=recv_sems.at[d - 1],
                    device_id=(my_pos + d,),
                    device_id_type=pl.DeviceIdType.MESH,
                ).start()

        local = lax.dot_general(
            tri, x_ref[:, :],
            (((1,), (0,)), ((), ())),
            preferred_element_type=jnp.float32,
        )

        offset = jnp.zeros((1, n), jnp.float32)
        for d in (1, 3, 2):

            @pl.when(my_pos >= d)
            def _():
                pltpu.make_async_remote_copy(
                    src_ref=comm_ref.at[0],
                    dst_ref=comm_ref.at[d],
                    send_sem=send_sems.at[d - 1],
                    recv_sem=recv_sems.at[d - 1],
                    device_id=(0,),
                    device_id_type=pl.DeviceIdType.MESH,
                ).wait_recv()

            offset = offset + jnp.where(
                my_pos >= d, comm_ref[d, :, :], 0.0
            )
        out_ref[:, :] = local + offset

        for d in range(1, N_DEV):

            @pl.when(my_pos + d <= N_DEV - 1)
            def _():
                pltpu.make_async_remote_copy(
                    src_ref=comm_ref.at[0],
                    dst_ref=comm_ref.at[d],
                    send_sem=send_sems.at[d - 1],
                    recv_sem=recv_sems.at[d - 1],
                    device_id=(my_pos + d,),
                    device_id_type=pl.DeviceIdType.MESH,
                ).wait_send()

    return pl.pallas_call(
        body,
        out_shape=jax.ShapeDtypeStruct((m_per, n), jnp.float32),
        in_specs=[pl.BlockSpec(memory_space=pltpu.VMEM)],
        out_specs=pl.BlockSpec(memory_space=pltpu.VMEM),
        scratch_shapes=[
            pltpu.VMEM((N_DEV, 1, n), jnp.float32),
            pltpu.SemaphoreType.DMA((N_DEV - 1,)),
            pltpu.SemaphoreType.DMA((N_DEV - 1,)),
        ],
        compiler_params=pltpu.CompilerParams(collective_id=0),
    )(x)
